# Initial kernel scaffold; baseline (speedup 1.0000x reference)
import jax
import jax.numpy as jnp
from jax.experimental import pallas as pl


def kernel(x, edge_index, W1_l, b1_l, W1_r, W2_l, b2_l, W2_r):
    raise NotImplementedError("write your pallas kernel here")


# speedup vs baseline: 6.7193x; 6.7193x over previous
import functools

import jax
import jax.numpy as jnp
from jax import lax
from jax.experimental import pallas as pl
from jax.experimental.pallas import tpu as pltpu
from jax.experimental.pallas import tpu_sc as plsc

N = 10000
F_IN = 128
HID = 128
C_OUT = 16
E = 320000

NC, NS = 2, 16
NW = NC * NS
CH = 128
JB = 79
EPW = JB * CH
EPAD = EPW * NW
NPAD = 10112
RPW = NPAD // NS

BLK = 400
GRID = N // BLK


def _make_segsum(d):
    mesh = plsc.VectorSubcoreMesh(core_axis_name="c", subcore_axis_name="s")

    @functools.partial(
        pl.kernel,
        mesh=mesh,
        compiler_params=pltpu.CompilerParams(use_tc_tiling_on_sc=(d >= 128)),
        out_type=jax.ShapeDtypeStruct((NC, NPAD, d), jnp.float32),
        scratch_types=[
            pltpu.VMEM((JB, CH), jnp.int32),
            pltpu.VMEM((JB, CH), jnp.int32),
            pltpu.VMEM((CH, d), jnp.float32),
            pltpu.VMEM_SHARED((NPAD, d), jnp.float32),
            pltpu.SemaphoreType.DMA,
        ],
    )
    def segsum(table, srcg, dstg, zrows, out, src_v, dst_v, rows_v, acc, sem):
        c = lax.axis_index("c")
        s = lax.axis_index("s")
        g = c * NS + s
        pltpu.sync_copy(srcg.at[g], src_v)
        pltpu.sync_copy(dstg.at[g], dst_v)
        pltpu.sync_copy(zrows, acc.at[pl.ds(s * RPW, RPW)])
        plsc.subcore_barrier()

        def body(j, carry):
            pltpu.async_copy(table.at[src_v.at[j]], rows_v, sem).wait()
            pltpu.sync_copy(rows_v, acc.at[dst_v.at[j]], add=True)
            return carry

        lax.fori_loop(0, JB, body, 0)
        plsc.subcore_barrier()
        pltpu.sync_copy(acc.at[pl.ds(s * RPW, RPW)], out.at[c, pl.ds(s * RPW, RPW)])

    return segsum


_segsum_h = _make_segsum(HID)
_segsum_c = _make_segsum(C_OUT)


def _mm_bias_body(x_ref, w_ref, b_ref, o_ref):
    o_ref[...] = (
        jnp.dot(x_ref[...], w_ref[...], preferred_element_type=jnp.float32)
        + b_ref[...]
    )


def _layer1_body(p_ref, xr_ref, w1l_ref, w2l_ref, h_ref, g_ref):
    agg = p_ref[0] + p_ref[1]
    h = jnp.maximum(
        jnp.dot(agg, w1l_ref[...], preferred_element_type=jnp.float32) + xr_ref[...],
        0.0,
    )
    h_ref[...] = h
    g_ref[...] = jnp.dot(h, w2l_ref[...], preferred_element_type=jnp.float32)


def _layer2_body(p_ref, h_ref, w2r_ref, b_ref, o_ref):
    o_ref[...] = (
        p_ref[0]
        + p_ref[1]
        + jnp.dot(h_ref[...], w2r_ref[...], preferred_element_type=jnp.float32)
        + b_ref[...]
    )


def kernel(x, edge_index, W1_l, b1_l, W1_r, W2_l, b2_l, W2_r):
    src = edge_index[0]
    dst = edge_index[1]
    pad = EPAD - E
    srcg = jnp.concatenate([src, jnp.zeros((pad,), jnp.int32)]).reshape(NW, JB, CH)
    dstg = jnp.concatenate([dst, jnp.full((pad,), N, jnp.int32)]).reshape(NW, JB, CH)
    zrows_h = jnp.zeros((RPW, HID), jnp.float32)
    zrows_c = jnp.zeros((RPW, C_OUT), jnp.float32)

    xr1 = pl.pallas_call(
        _mm_bias_body,
        grid=(GRID,),
        in_specs=[
            pl.BlockSpec((BLK, F_IN), lambda i: (i, 0)),
            pl.BlockSpec((F_IN, HID), lambda i: (0, 0)),
            pl.BlockSpec((1, HID), lambda i: (0, 0)),
        ],
        out_specs=pl.BlockSpec((BLK, HID), lambda i: (i, 0)),
        out_shape=jax.ShapeDtypeStruct((N, HID), jnp.float32),
    )(x, W1_r, b1_l.reshape(1, HID))

    p1 = _segsum_h(x, srcg, dstg, zrows_h)

    h, g = pl.pallas_call(
        _layer1_body,
        grid=(GRID,),
        in_specs=[
            pl.BlockSpec((NC, BLK, HID), lambda i: (0, i, 0)),
            pl.BlockSpec((BLK, HID), lambda i: (i, 0)),
            pl.BlockSpec((F_IN, HID), lambda i: (0, 0)),
            pl.BlockSpec((HID, C_OUT), lambda i: (0, 0)),
        ],
        out_specs=[
            pl.BlockSpec((BLK, HID), lambda i: (i, 0)),
            pl.BlockSpec((BLK, C_OUT), lambda i: (i, 0)),
        ],
        out_shape=[
            jax.ShapeDtypeStruct((N, HID), jnp.float32),
            jax.ShapeDtypeStruct((N, C_OUT), jnp.float32),
        ],
    )(p1, xr1, W1_l, W2_l)

    p2 = _segsum_c(g, srcg, dstg, zrows_c)

    out = pl.pallas_call(
        _layer2_body,
        grid=(GRID,),
        in_specs=[
            pl.BlockSpec((NC, BLK, C_OUT), lambda i: (0, i, 0)),
            pl.BlockSpec((BLK, HID), lambda i: (i, 0)),
            pl.BlockSpec((HID, C_OUT), lambda i: (0, 0)),
            pl.BlockSpec((1, C_OUT), lambda i: (0, 0)),
        ],
        out_specs=pl.BlockSpec((BLK, C_OUT), lambda i: (i, 0)),
        out_shape=jax.ShapeDtypeStruct((N, C_OUT), jnp.float32),
    )(p2, h, W2_r, b2_l.reshape(1, C_OUT))
    return out
